# baseline (speedup 1.0000x reference)
import functools

import jax
import jax.numpy as jnp
from jax.experimental import pallas as pl

N = 10000
E = 320000
D = 128
NT = 17
STEPS = 2

_DN = (((1,), (0,)), ((), ()))


def _dot(a, w):
    return jax.lax.dot_general(a, w, _DN, preferred_element_type=jnp.float32)


def _bf(x):
    return x.astype(jnp.bfloat16)


def _node_emb_body(x_ref, w1_ref, b1_ref, w2_ref, b2_ref, o_ref):
    h = _bf(jax.nn.relu(_dot(x_ref[...], w1_ref[...]) + b1_ref[...]))
    o_ref[...] = _bf(jax.nn.relu(_dot(h, _bf(w2_ref[...])) + b2_ref[...]))


def _node_emb(x, w1, b1, w2, b2):
    return pl.pallas_call(
        _node_emb_body,
        out_shape=jax.ShapeDtypeStruct((N, D), jnp.bfloat16),
    )(x, w1, b1, w2, b2)


def _edge_emb_body(a_ref, w1_ref, b1_ref, w2_ref, b2_ref, o_ref):
    h = _bf(jax.nn.relu(_dot(a_ref[...], w1_ref[...]) + b1_ref[...]))
    o_ref[...] = _bf(jax.nn.relu(_dot(h, _bf(w2_ref[...])) + b2_ref[...]))


def _edge_emb(ea, w1, b1, w2, b2, block=8000):
    return pl.pallas_call(
        _edge_emb_body,
        grid=(E // block,),
        in_specs=[pl.BlockSpec((block, 16), lambda i: (i, 0)),
                  pl.BlockSpec((16, D), lambda i: (0, 0)),
                  pl.BlockSpec((D,), lambda i: (0,)),
                  pl.BlockSpec((D, D), lambda i: (0, 0)),
                  pl.BlockSpec((D,), lambda i: (0,))],
        out_specs=pl.BlockSpec((block, D), lambda i: (i, 0)),
        out_shape=jax.ShapeDtypeStruct((E, D), jnp.bfloat16),
    )(ea, w1, b1, w2, b2)


def _edge_mlp_body(gs_ref, gd_ref, ef_ref, w1_ref, b1_ref, w2_ref, b2_ref, o_ref):
    e = ef_ref[...]
    if e.dtype != jnp.bfloat16:
        e = _bf(e)
    cc = jnp.concatenate([gs_ref[...], gd_ref[...], e], axis=1)
    h = _bf(jax.nn.relu(_dot(cc, _bf(w1_ref[...])) + b1_ref[...]))
    o_ref[...] = jax.nn.relu(_dot(h, _bf(w2_ref[...])) + b2_ref[...])


def _edge_mlp(gs, gd, ef, w1, b1, w2, b2, block=4000):
    eft = jnp.float32 if ef.dtype == jnp.float32 else jnp.bfloat16
    return pl.pallas_call(
        _edge_mlp_body,
        grid=(E // block,),
        in_specs=[pl.BlockSpec((block, D), lambda i: (i, 0)),
                  pl.BlockSpec((block, D), lambda i: (i, 0)),
                  pl.BlockSpec((block, D), lambda i: (i, 0)),
                  pl.BlockSpec((3 * D, D), lambda i: (0, 0)),
                  pl.BlockSpec((D,), lambda i: (0,)),
                  pl.BlockSpec((D, D), lambda i: (0, 0)),
                  pl.BlockSpec((D,), lambda i: (0,))],
        out_specs=pl.BlockSpec((block, D), lambda i: (i, 0)),
        out_shape=jax.ShapeDtypeStruct((E, D), jnp.float32),
    )(gs, gd, ef, w1, b1, w2, b2)


def _edge_head_body(ef_ref, w1_ref, b1_ref, w2_ref, b2_ref, o_ref):
    h = jax.nn.relu(_dot(ef_ref[...], w1_ref[...]) + b1_ref[...])
    o_ref[...] = jnp.reshape(_dot(h, w2_ref[...]) + b2_ref[...], (-1,))


def _edge_head(ef, w1, b1, w2, b2, block=512):
    return pl.pallas_call(
        _edge_head_body,
        grid=(E // block,),
        in_specs=[pl.BlockSpec((block, D), lambda i: (i, 0)),
                  pl.BlockSpec((D, D), lambda i: (0, 0)),
                  pl.BlockSpec((D,), lambda i: (0,)),
                  pl.BlockSpec((D, 1), lambda i: (0, 0)),
                  pl.BlockSpec((1,), lambda i: (0,))],
        out_specs=pl.BlockSpec((block,), lambda i: (i,)),
        out_shape=jax.ShapeDtypeStruct((E,), jnp.float32),
    )(ef, w1, b1, w2, b2)


def _node_mlp_body(nf_ref, agg_ref, w_ref, b_ref, o_ref, *, out_bf16):
    cc = jnp.concatenate([nf_ref[...], _bf(agg_ref[...])], axis=1)
    o = jax.nn.relu(_dot(cc, _bf(w_ref[...])) + b_ref[...])
    o_ref[...] = _bf(o) if out_bf16 else o


def _node_mlp(nf, agg, w, b, out_bf16):
    return pl.pallas_call(
        functools.partial(_node_mlp_body, out_bf16=out_bf16),
        out_shape=jax.ShapeDtypeStruct(
            (N, D), jnp.bfloat16 if out_bf16 else jnp.float32),
    )(nf, agg, w, b)


def _heads_body(nf_ref, wn1_ref, bn1_ref, wn2_ref, bn2_ref,
                wc1_ref, bc1_ref, wc2_ref, bc2_ref,
                we_ref, be_ref,
                pn_ref, pc_ref, ty_ref, ne_ref):
    nf = nf_ref[...]
    hn = jax.nn.relu(_dot(nf, wn1_ref[...]) + bn1_ref[...])
    pn_ref[...] = _dot(hn, wn2_ref[...]) + bn2_ref[...]
    hc = jax.nn.relu(_dot(nf, wc1_ref[...]) + bc1_ref[...])
    pc = _dot(hc, wc2_ref[...]) + bc2_ref[...]
    pc_ref[...] = pc
    mx = jnp.max(pc, axis=1, keepdims=True)
    iota = jax.lax.broadcasted_iota(jnp.int32, pc.shape, 1)
    ty = jnp.min(jnp.where(pc == mx, iota, NT), axis=1, keepdims=True)
    ty_ref[...] = ty
    ne_ref[...] = _dot(nf, we_ref[...]) + be_ref[...]


def _heads(nf, p):
    (wn1, bn1), (wn2, bn2) = p['node_cls']
    (wc1, bc1), (wc2, bc2) = p['cls']
    we, be = p['edge_const']
    return pl.pallas_call(
        _heads_body,
        out_shape=(jax.ShapeDtypeStruct((N, 1), jnp.float32),
                   jax.ShapeDtypeStruct((N, NT), jnp.float32),
                   jax.ShapeDtypeStruct((N, 1), jnp.int32),
                   jax.ShapeDtypeStruct((N, D), jnp.float32)),
    )(nf, wn1, bn1, wn2, bn2, wc1, bc1, wc2, bc2, we, be)


def _score_body(a_ref, b_ref, o_ref):
    p = a_ref[...] * b_ref[...]
    acc = p[:, 0:8]
    for j in range(1, 16):
        acc = acc + p[:, 8 * j:8 * j + 8]
    t = acc[:, 0:4] + acc[:, 4:8]
    t = t[:, 0:2] + t[:, 2:4]
    o_ref[...] = jnp.reshape(t[:, 0:1] + t[:, 1:2], (-1,))


def _scores(a, b, block=512):
    return pl.pallas_call(
        _score_body,
        grid=(E // block,),
        in_specs=[pl.BlockSpec((block, D), lambda i: (i, 0)),
                  pl.BlockSpec((block, D), lambda i: (i, 0))],
        out_specs=pl.BlockSpec((block,), lambda i: (i,)),
        out_shape=jax.ShapeDtypeStruct((E,), jnp.float32),
    )(a, b)


def kernel(x, edge_attr, params, edge_index, node_types):
    src = edge_index[0]
    dst = edge_index[1]
    (wne1, bne1), (wne2, bne2) = params['node_emb']
    (wee1, bee1), (wee2, bee2) = params['edge_emb']
    (w1, b1), (w2, b2) = params['mpn_edge']
    (wn, bn), = params['mpn_node']

    nf = _node_emb(x, wne1, bne1, wne2, bne2)
    ef = _edge_emb(edge_attr, wee1, bee1, wee2, bee2)

    for step in range(STEPS):
        gs = jnp.take(nf, src, axis=0)
        gd = jnp.take(nf, dst, axis=0)
        ef = _edge_mlp(gs, gd, ef, w1, b1, w2, b2)
        agg = jax.ops.segment_sum(ef, dst, num_segments=N)
        nf = _node_mlp(nf, agg, wn, bn, out_bf16=(step == 0))

    pn, pc, ty, ne = _heads(nf, params)
    pred_node = pn[:, 0]
    pred_class = pc
    (we1, be1), (we2, be2) = params['edge_cls']
    edge_pred = _edge_head(ef, we1, be1, we2, be2)

    nes = jnp.take(ne, src, axis=0)
    ned = jnp.take(ne, dst, axis=0)
    scores = _scores(nes, ned)

    source_types = ty[:, 0][src]
    seg = dst * NT + source_types
    mx = jax.ops.segment_max(scores, seg, num_segments=N * NT)
    mx = jnp.where(jnp.isfinite(mx), mx, 0.0)
    ex = jnp.exp(scores - mx[seg])
    den = jax.ops.segment_sum(ex, seg, num_segments=N * NT)
    den = jnp.where(den > 0, den, 1.0)
    pred_edge = (ex / den[seg]) * jax.nn.sigmoid(edge_pred)
    return ([pred_edge], [pred_node], [pred_class], nf, ef)


# speedup vs baseline: 2.1566x; 2.1566x over previous
import functools

import jax
import jax.numpy as jnp
from jax import lax
from jax.experimental import pallas as pl
from jax.experimental.pallas import tpu as pltpu
from jax.experimental.pallas import tpu_sc as plsc

N = 10000
E = 320000
D = 128
NT = 17
STEPS = 2

NW = 32
EPW = E // NW
TID_P = 85120
NSEG_P = 2 * TID_P

_mesh = functools.partial(plsc.VectorSubcoreMesh,
                          core_axis_name="c", subcore_axis_name="s")


def _wid():
    return lax.axis_index("s") * 2 + lax.axis_index("c")

_DN = (((1,), (0,)), ((), ()))


def _dot(a, w):
    return jax.lax.dot_general(a, w, _DN, preferred_element_type=jnp.float32)


def _bf(x):
    return x.astype(jnp.bfloat16)


def _node_emb_body(x_ref, w1_ref, b1_ref, w2_ref, b2_ref, o_ref):
    h = _bf(jax.nn.relu(_dot(x_ref[...], w1_ref[...]) + b1_ref[...]))
    o_ref[...] = _bf(jax.nn.relu(_dot(h, _bf(w2_ref[...])) + b2_ref[...])).astype(jnp.float32)


def _node_emb(x, w1, b1, w2, b2):
    return pl.pallas_call(
        _node_emb_body,
        out_shape=jax.ShapeDtypeStruct((N, D), jnp.float32),
    )(x, w1, b1, w2, b2)


def _edge_emb_body(a_ref, w1_ref, b1_ref, w2_ref, b2_ref, o_ref):
    h = _bf(jax.nn.relu(_dot(a_ref[...], w1_ref[...]) + b1_ref[...]))
    o_ref[...] = _bf(jax.nn.relu(_dot(h, _bf(w2_ref[...])) + b2_ref[...]))


def _edge_emb(ea, w1, b1, w2, b2, block=8000):
    return pl.pallas_call(
        _edge_emb_body,
        grid=(E // block,),
        in_specs=[pl.BlockSpec((block, 16), lambda i: (i, 0)),
                  pl.BlockSpec((16, D), lambda i: (0, 0)),
                  pl.BlockSpec((D,), lambda i: (0,)),
                  pl.BlockSpec((D, D), lambda i: (0, 0)),
                  pl.BlockSpec((D,), lambda i: (0,))],
        out_specs=pl.BlockSpec((block, D), lambda i: (i, 0)),
        out_shape=jax.ShapeDtypeStruct((E, D), jnp.bfloat16),
    )(ea, w1, b1, w2, b2)


def _edge_mlp_body(gs_ref, gd_ref, ef_ref, w1_ref, b1_ref, w2_ref, b2_ref, o_ref):
    e = ef_ref[...]
    if e.dtype != jnp.bfloat16:
        e = _bf(e)
    cc = jnp.concatenate([_bf(gs_ref[...]), _bf(gd_ref[...]), e], axis=1)
    h = _bf(jax.nn.relu(_dot(cc, _bf(w1_ref[...])) + b1_ref[...]))
    o_ref[...] = jax.nn.relu(_dot(h, _bf(w2_ref[...])) + b2_ref[...])


def _edge_mlp(gs, gd, ef, w1, b1, w2, b2, block=4000):
    eft = jnp.float32 if ef.dtype == jnp.float32 else jnp.bfloat16
    return pl.pallas_call(
        _edge_mlp_body,
        grid=(E // block,),
        in_specs=[pl.BlockSpec((block, D), lambda i: (i, 0)),
                  pl.BlockSpec((block, D), lambda i: (i, 0)),
                  pl.BlockSpec((block, D), lambda i: (i, 0)),
                  pl.BlockSpec((3 * D, D), lambda i: (0, 0)),
                  pl.BlockSpec((D,), lambda i: (0,)),
                  pl.BlockSpec((D, D), lambda i: (0, 0)),
                  pl.BlockSpec((D,), lambda i: (0,))],
        out_specs=pl.BlockSpec((block, D), lambda i: (i, 0)),
        out_shape=jax.ShapeDtypeStruct((E, D), jnp.float32),
    )(gs, gd, ef, w1, b1, w2, b2)


def _edge_head_body(ef_ref, w1_ref, b1_ref, w2_ref, b2_ref, o_ref):
    h = jax.nn.relu(_dot(ef_ref[...], w1_ref[...]) + b1_ref[...])
    o_ref[...] = jnp.reshape(_dot(h, w2_ref[...]) + b2_ref[...], (-1,))


def _edge_head(ef, w1, b1, w2, b2, block=512):
    return pl.pallas_call(
        _edge_head_body,
        grid=(E // block,),
        in_specs=[pl.BlockSpec((block, D), lambda i: (i, 0)),
                  pl.BlockSpec((D, D), lambda i: (0, 0)),
                  pl.BlockSpec((D,), lambda i: (0,)),
                  pl.BlockSpec((D, 1), lambda i: (0, 0)),
                  pl.BlockSpec((1,), lambda i: (0,))],
        out_specs=pl.BlockSpec((block,), lambda i: (i,)),
        out_shape=jax.ShapeDtypeStruct((E,), jnp.float32),
    )(ef, w1, b1, w2, b2)


def _node_mlp_body(nf_ref, agg_ref, w_ref, b_ref, o_ref, *, out_bf16):
    cc = jnp.concatenate([_bf(nf_ref[...]), _bf(agg_ref[...])], axis=1)
    o = jax.nn.relu(_dot(cc, _bf(w_ref[...])) + b_ref[...])
    o_ref[...] = _bf(o).astype(jnp.float32) if out_bf16 else o


def _node_mlp(nf, agg, w, b, out_bf16):
    return pl.pallas_call(
        functools.partial(_node_mlp_body, out_bf16=out_bf16),
        out_shape=jax.ShapeDtypeStruct((N, D), jnp.float32),
    )(nf, agg, w, b)


def _heads_body(nf_ref, wn1_ref, bn1_ref, wn2_ref, bn2_ref,
                wc1_ref, bc1_ref, wc2_ref, bc2_ref,
                we_ref, be_ref,
                pn_ref, pc_ref, ty_ref, ne_ref):
    nf = nf_ref[...]
    hn = jax.nn.relu(_dot(nf, wn1_ref[...]) + bn1_ref[...])
    pn_ref[...] = _dot(hn, wn2_ref[...]) + bn2_ref[...]
    hc = jax.nn.relu(_dot(nf, wc1_ref[...]) + bc1_ref[...])
    pc = _dot(hc, wc2_ref[...]) + bc2_ref[...]
    pc_ref[...] = pc
    mx = jnp.max(pc, axis=1, keepdims=True)
    iota = jax.lax.broadcasted_iota(jnp.int32, pc.shape, 1)
    ty = jnp.min(jnp.where(pc == mx, iota, NT), axis=1, keepdims=True)
    ty_ref[...] = ty
    ne_ref[...] = _dot(nf, we_ref[...]) + be_ref[...]


def _heads(nf, p):
    (wn1, bn1), (wn2, bn2) = p['node_cls']
    (wc1, bc1), (wc2, bc2) = p['cls']
    we, be = p['edge_const']
    return pl.pallas_call(
        _heads_body,
        out_shape=(jax.ShapeDtypeStruct((N, 1), jnp.float32),
                   jax.ShapeDtypeStruct((N, NT), jnp.float32),
                   jax.ShapeDtypeStruct((N, 1), jnp.int32),
                   jax.ShapeDtypeStruct((N, D), jnp.float32)),
    )(nf, wn1, bn1, wn2, bn2, wc1, bc1, wc2, bc2, we, be)


def _score_body(a_ref, b_ref, o_ref):
    p = a_ref[...] * b_ref[...]
    acc = p[:, 0:8]
    for j in range(1, 16):
        acc = acc + p[:, 8 * j:8 * j + 8]
    t = acc[:, 0:4] + acc[:, 4:8]
    t = t[:, 0:2] + t[:, 2:4]
    o_ref[...] = jnp.reshape(t[:, 0:1] + t[:, 1:2], (-1,))


def _scores(a, b, block=512):
    return pl.pallas_call(
        _score_body,
        grid=(E // block,),
        in_specs=[pl.BlockSpec((block, D), lambda i: (i, 0)),
                  pl.BlockSpec((block, D), lambda i: (i, 0))],
        out_specs=pl.BlockSpec((block,), lambda i: (i,)),
        out_shape=jax.ShapeDtypeStruct((E,), jnp.float32),
    )(a, b)


def _sc_gather_rows(table, idx, chunk=400):
    n_rows, r = table.shape
    n_idx = idx.shape[0]
    per_w = n_idx // NW
    n_ch = per_w // chunk

    @functools.partial(
        pl.kernel, mesh=_mesh(),
        compiler_params=pltpu.CompilerParams(needs_layout_passes=False),
        out_type=jax.ShapeDtypeStruct((n_idx, r), table.dtype),
        scratch_types=[pltpu.VMEM((chunk,), jnp.int32),
                       pltpu.VMEM((chunk, r), table.dtype),
                       pltpu.SemaphoreType.DMA],
    )
    def k(tb, ix, out, idx_v, rows_v, sem):
        base = pl.multiple_of(_wid() * per_w, 8)

        def body(i, carry):
            off = base + i * chunk
            pltpu.sync_copy(ix.at[pl.ds(off, chunk)], idx_v)
            pltpu.async_copy(tb.at[idx_v], rows_v, sem).wait()
            pltpu.sync_copy(rows_v, out.at[pl.ds(off, chunk)])
            return carry

        lax.fori_loop(0, n_ch, body, 0)

    return k(table, idx)


def _sc_seg_kernel(ty, src, dst, chunk=2000):
    n_ch = EPW // chunk

    @functools.partial(
        pl.kernel, mesh=_mesh(),
        compiler_params=pltpu.CompilerParams(needs_layout_passes=False),
        out_type=jax.ShapeDtypeStruct((E,), jnp.int32),
        scratch_types=[pltpu.VMEM((chunk,), jnp.int32),
                       pltpu.VMEM((chunk,), jnp.int32),
                       pltpu.VMEM((chunk,), jnp.int32),
                       pltpu.VMEM((chunk,), jnp.int32),
                       pltpu.SemaphoreType.DMA],
    )
    def k(ty_h, src_h, dst_h, out_h, sv, dv, tv, ov, sem):
        base = pl.multiple_of(_wid() * EPW, 8)

        def body(i, carry):
            off = base + i * chunk
            pltpu.sync_copy(src_h.at[pl.ds(off, chunk)], sv)
            pltpu.sync_copy(dst_h.at[pl.ds(off, chunk)], dv)
            pltpu.async_copy(ty_h.at[sv], tv, sem).wait()

            def vbody(j, c):
                d16 = dv[pl.ds(j * 16, 16)]
                t16 = tv[pl.ds(j * 16, 16)]
                seg = d16 * NT + t16
                tid = (seg & 1) * TID_P + lax.shift_right_logical(seg, 1)
                ov[pl.ds(j * 16, 16)] = tid
                return c

            lax.fori_loop(0, chunk // 16, vbody, 0)
            pltpu.sync_copy(ov, out_h.at[pl.ds(off, chunk)])
            return carry

        lax.fori_loop(0, n_ch, body, 0)

    return k(ty, src, dst)


_NEG = -3.0e38


def _sc_max_partial(tidx, scores, chunk=2000):
    n_ch = (E // 16) // chunk

    @functools.partial(
        pl.kernel, mesh=_mesh(),
        compiler_params=pltpu.CompilerParams(needs_layout_passes=False),
        out_type=jax.ShapeDtypeStruct((NW * TID_P,), jnp.float32),
        scratch_types=[pltpu.VMEM((TID_P,), jnp.float32),
                       pltpu.VMEM((chunk,), jnp.int32),
                       pltpu.VMEM((chunk,), jnp.float32),
                       pltpu.SemaphoreType.DMA],
    )
    def k(tid_h, sc_h, out_h, tbl, tv, sv, sem):
        w = _wid()
        par = w % 2
        base = pl.multiple_of((w // 2) * (E // 16), 8)
        neg = jnp.full((16,), _NEG, jnp.float32)

        def initb(i, c):
            tbl[pl.ds(i * 16, 16)] = neg
            return c

        lax.fori_loop(0, TID_P // 16, initb, 0)
        lo = par * TID_P

        def body(i, carry):
            off = base + i * chunk
            pltpu.sync_copy(tid_h.at[pl.ds(off, chunk)], tv)
            pltpu.sync_copy(sc_h.at[pl.ds(off, chunk)], sv)

            def vbody(j, c):
                tid = tv[pl.ds(j * 16, 16)]
                val = sv[pl.ds(j * 16, 16)]
                iota16 = lax.iota(jnp.int32, 16)
                loc = tid - lo
                act = (loc >= 0) & (loc < TID_P)
                key = jnp.where(act, loc, TID_P)
                k, v = plsc.sort_key_val(key, val)
                for sft in (1, 2, 4, 8):
                    idxs = jnp.maximum(iota16 - sft, 0)
                    kk = k.at[idxs].get(mode="promise_in_bounds")
                    vv = v.at[idxs].get(mode="promise_in_bounds")
                    same = (kk == k) & (iota16 >= sft)
                    v = jnp.where(same, jnp.maximum(v, vv), v)
                kup = k.at[jnp.minimum(iota16 + 1, 15)].get(
                    mode="promise_in_bounds")
                m = ((k != kup) | (iota16 == 15)) & (k < TID_P)
                kc = jnp.where(m, k, 0)
                cur = plsc.load_gather(tbl, [kc], mask=m)
                new = jnp.maximum(cur, v)
                plsc.store_scatter(tbl, [kc], new, mask=m)
                return c

            lax.fori_loop(0, chunk // 16, vbody, 0)
            return carry

        lax.fori_loop(0, n_ch, body, 0)
        pltpu.sync_copy(tbl, out_h.at[pl.ds(pl.multiple_of(w * TID_P, 8), TID_P)])

    return k(tidx, scores)


def _sc_max_merge(partials):
    sl = TID_P // 16

    @functools.partial(
        pl.kernel, mesh=_mesh(),
        compiler_params=pltpu.CompilerParams(needs_layout_passes=False),
        out_type=jax.ShapeDtypeStruct((NSEG_P,), jnp.float32),
        scratch_types=[pltpu.VMEM((sl,), jnp.float32),
                       pltpu.VMEM((sl,), jnp.float32)],
    )
    def k(pt_h, out_h, acc, buf):
        w = _wid()
        par = w // 16
        rr = w % 16
        off = rr * sl
        pltpu.sync_copy(pt_h.at[pl.ds(pl.multiple_of(par * TID_P + off, 8), sl)], acc)

        def body(t, c):
            pltpu.sync_copy(pt_h.at[pl.ds(pl.multiple_of((2 * t + par) * TID_P + off, 8), sl)], buf)

            def vbody(j, cc):
                acc[pl.ds(j * 16, 16)] = jnp.maximum(acc[pl.ds(j * 16, 16)],
                                                     buf[pl.ds(j * 16, 16)])
                return cc

            lax.fori_loop(0, sl // 16, vbody, 0)
            return c

        lax.fori_loop(1, 16, body, 0)
        pltpu.sync_copy(acc, out_h.at[pl.ds(pl.multiple_of(par * TID_P + off, 8), sl)])

    return k(partials)


def _sc_ex_den(tidx, scores, mx_pm, chunk=2000):
    n_ch = EPW // chunk

    @functools.partial(
        pl.kernel, mesh=_mesh(),
        compiler_params=pltpu.CompilerParams(needs_layout_passes=False),
        out_type=(jax.ShapeDtypeStruct((E,), jnp.float32),
                  jax.ShapeDtypeStruct((2 * NSEG_P,), jnp.float32)),
        scratch_types=[pltpu.VMEM((chunk,), jnp.int32),
                       pltpu.VMEM((chunk,), jnp.float32),
                       pltpu.VMEM((chunk,), jnp.float32),
                       pltpu.VMEM((chunk,), jnp.float32),
                       pltpu.VMEM((NSEG_P // 16,), jnp.float32),
                       pltpu.VMEM_SHARED((NSEG_P,), jnp.float32),
                       pltpu.SemaphoreType.DMA],
    )
    def k(tid_h, sc_h, mx_h, ex_h, den_h, tv, sv, mv, ev, zv, shared, sem):
        w = _wid()
        core = lax.axis_index("c")
        sub = lax.axis_index("s")
        base = pl.multiple_of(w * EPW, 8)
        zn = NSEG_P // 16
        zero = jnp.zeros((16,), jnp.float32)

        def zb(i, c):
            zv[pl.ds(i * 16, 16)] = zero
            return c

        lax.fori_loop(0, zn // 16, zb, 0)
        pltpu.sync_copy(zv, shared.at[pl.ds(pl.multiple_of(sub * zn, 8), zn)])
        plsc.subcore_barrier()

        def body(i, carry):
            off = base + i * chunk
            pltpu.sync_copy(tid_h.at[pl.ds(off, chunk)], tv)
            pltpu.sync_copy(sc_h.at[pl.ds(off, chunk)], sv)
            pltpu.async_copy(mx_h.at[tv], mv, sem).wait()

            def vbody(j, c):
                s16 = sv[pl.ds(j * 16, 16)]
                m16 = mv[pl.ds(j * 16, 16)]
                ev[pl.ds(j * 16, 16)] = jnp.exp(s16 - m16)
                return c

            lax.fori_loop(0, chunk // 16, vbody, 0)
            pltpu.sync_copy(ev, ex_h.at[pl.ds(off, chunk)])
            pltpu.sync_copy(ev, shared.at[tv], add=True)
            return carry

        lax.fori_loop(0, n_ch, body, 0)
        plsc.subcore_barrier()
        pltpu.sync_copy(shared.at[pl.ds(pl.multiple_of(sub * zn, 8), zn)], zv)
        pltpu.sync_copy(zv, den_h.at[pl.ds(pl.multiple_of(core * NSEG_P + sub * zn, 8), zn)])

    return k(tidx, scores, mx_pm)


def _sc_final(tidx, ex, sig, den0, den1, chunk=2000):
    n_ch = EPW // chunk

    @functools.partial(
        pl.kernel, mesh=_mesh(),
        compiler_params=pltpu.CompilerParams(needs_layout_passes=False),
        out_type=jax.ShapeDtypeStruct((E,), jnp.float32),
        scratch_types=[pltpu.VMEM((chunk,), jnp.int32),
                       pltpu.VMEM((chunk,), jnp.float32),
                       pltpu.VMEM((chunk,), jnp.float32),
                       pltpu.VMEM((chunk,), jnp.float32),
                       pltpu.VMEM((chunk,), jnp.float32),
                       pltpu.VMEM((chunk,), jnp.float32),
                       pltpu.SemaphoreType.DMA],
    )
    def k(tid_h, ex_h, sg_h, d0_h, d1_h, out_h, tv, evv, gv, d0v, d1v, ov, sem):
        base = pl.multiple_of(_wid() * EPW, 8)

        def body(i, carry):
            off = base + i * chunk
            pltpu.sync_copy(tid_h.at[pl.ds(off, chunk)], tv)
            pltpu.sync_copy(ex_h.at[pl.ds(off, chunk)], evv)
            pltpu.sync_copy(sg_h.at[pl.ds(off, chunk)], gv)
            pltpu.async_copy(d0_h.at[tv], d0v, sem).wait()
            pltpu.async_copy(d1_h.at[tv], d1v, sem).wait()

            def vbody(j, c):
                e16 = evv[pl.ds(j * 16, 16)]
                g16 = gv[pl.ds(j * 16, 16)]
                d16 = d0v[pl.ds(j * 16, 16)] + d1v[pl.ds(j * 16, 16)]
                ov[pl.ds(j * 16, 16)] = e16 / d16 * g16
                return c

            lax.fori_loop(0, chunk // 16, vbody, 0)
            pltpu.sync_copy(ov, out_h.at[pl.ds(off, chunk)])
            return carry

        lax.fori_loop(0, n_ch, body, 0)

    return k(tidx, ex, sig, den0, den1)


def kernel(x, edge_attr, params, edge_index, node_types):
    src = edge_index[0]
    dst = edge_index[1]
    (wne1, bne1), (wne2, bne2) = params['node_emb']
    (wee1, bee1), (wee2, bee2) = params['edge_emb']
    (w1, b1), (w2, b2) = params['mpn_edge']
    (wn, bn), = params['mpn_node']

    nf = _node_emb(x, wne1, bne1, wne2, bne2)
    ef = _edge_emb(edge_attr, wee1, bee1, wee2, bee2)

    for step in range(STEPS):
        gs = _sc_gather_rows(nf, src)
        gd = _sc_gather_rows(nf, dst)
        ef = _edge_mlp(gs, gd, ef, w1, b1, w2, b2)
        agg = jax.ops.segment_sum(ef, dst, num_segments=N)
        nf = _node_mlp(nf, agg, wn, bn, out_bf16=(step == 0))

    pn, pc, ty, ne = _heads(nf, params)
    pred_node = pn[:, 0]
    pred_class = pc
    (we1, be1), (we2, be2) = params['edge_cls']
    sig = jax.nn.sigmoid(_edge_head(ef, we1, be1, we2, be2))

    nes = _sc_gather_rows(ne, src)
    ned = _sc_gather_rows(ne, dst)
    scores = _scores(nes, ned)

    tidx = _sc_seg_kernel(ty[:, 0], src, dst)
    parts = _sc_max_partial(tidx, scores)
    mx_pm = _sc_max_merge(parts)
    ex, den = _sc_ex_den(tidx, scores, mx_pm)
    pred_edge = _sc_final(tidx, ex, sig, den[:NSEG_P], den[NSEG_P:])
    return ([pred_edge], [pred_node], [pred_class], nf, ef)


# speedup vs baseline: 2.2076x; 1.0236x over previous
import functools

import jax
import jax.numpy as jnp
from jax import lax
from jax.experimental import pallas as pl
from jax.experimental.pallas import tpu as pltpu
from jax.experimental.pallas import tpu_sc as plsc

N = 10000
E = 320000
D = 128
NT = 17
STEPS = 2

NW = 32
EPW = E // NW
TID_P = 85120
NSEG_P = 2 * TID_P

_mesh = functools.partial(plsc.VectorSubcoreMesh,
                          core_axis_name="c", subcore_axis_name="s")


def _wid():
    return lax.axis_index("s") * 2 + lax.axis_index("c")

_DN = (((1,), (0,)), ((), ()))


def _dot(a, w):
    return jax.lax.dot_general(a, w, _DN, preferred_element_type=jnp.float32)


def _bf(x):
    return x.astype(jnp.bfloat16)


def _node_emb_body(x_ref, w1_ref, b1_ref, w2_ref, b2_ref, o_ref):
    h = _bf(jax.nn.relu(_dot(x_ref[...], w1_ref[...]) + b1_ref[...]))
    o_ref[...] = _bf(jax.nn.relu(_dot(h, _bf(w2_ref[...])) + b2_ref[...])).astype(jnp.float32)


def _node_emb(x, w1, b1, w2, b2):
    return pl.pallas_call(
        _node_emb_body,
        out_shape=jax.ShapeDtypeStruct((N, D), jnp.float32),
    )(x, w1, b1, w2, b2)


def _edge_emb_body(a_ref, w1_ref, b1_ref, w2_ref, b2_ref, o_ref):
    h = _bf(jax.nn.relu(_dot(a_ref[...], w1_ref[...]) + b1_ref[...]))
    o_ref[...] = _bf(jax.nn.relu(_dot(h, _bf(w2_ref[...])) + b2_ref[...]))


def _edge_emb(ea, w1, b1, w2, b2, block=8000):
    return pl.pallas_call(
        _edge_emb_body,
        grid=(E // block,),
        in_specs=[pl.BlockSpec((block, 16), lambda i: (i, 0)),
                  pl.BlockSpec((16, D), lambda i: (0, 0)),
                  pl.BlockSpec((D,), lambda i: (0,)),
                  pl.BlockSpec((D, D), lambda i: (0, 0)),
                  pl.BlockSpec((D,), lambda i: (0,))],
        out_specs=pl.BlockSpec((block, D), lambda i: (i, 0)),
        out_shape=jax.ShapeDtypeStruct((E, D), jnp.bfloat16),
    )(ea, w1, b1, w2, b2)


def _edge_mlp_body(gs_ref, gd_ref, ef_ref, w1_ref, b1_ref, w2_ref, b2_ref, o_ref):
    e = ef_ref[...]
    if e.dtype != jnp.bfloat16:
        e = _bf(e)
    cc = jnp.concatenate([_bf(gs_ref[...]), _bf(gd_ref[...]), e], axis=1)
    h = _bf(jax.nn.relu(_dot(cc, _bf(w1_ref[...])) + b1_ref[...]))
    o_ref[...] = jax.nn.relu(_dot(h, _bf(w2_ref[...])) + b2_ref[...])


def _edge_mlp(gs, gd, ef, w1, b1, w2, b2, block=8000):
    eft = jnp.float32 if ef.dtype == jnp.float32 else jnp.bfloat16
    return pl.pallas_call(
        _edge_mlp_body,
        grid=(E // block,),
        in_specs=[pl.BlockSpec((block, D), lambda i: (i, 0)),
                  pl.BlockSpec((block, D), lambda i: (i, 0)),
                  pl.BlockSpec((block, D), lambda i: (i, 0)),
                  pl.BlockSpec((3 * D, D), lambda i: (0, 0)),
                  pl.BlockSpec((D,), lambda i: (0,)),
                  pl.BlockSpec((D, D), lambda i: (0, 0)),
                  pl.BlockSpec((D,), lambda i: (0,))],
        out_specs=pl.BlockSpec((block, D), lambda i: (i, 0)),
        out_shape=jax.ShapeDtypeStruct((E, D), jnp.float32),
    )(gs, gd, ef, w1, b1, w2, b2)


def _edge_head_body(ef_ref, w1_ref, b1_ref, w2_ref, b2_ref, o_ref):
    h = jax.nn.relu(_dot(ef_ref[...], w1_ref[...]) + b1_ref[...])
    o_ref[...] = jnp.reshape(_dot(h, w2_ref[...]) + b2_ref[...], (-1,))


def _edge_head(ef, w1, b1, w2, b2, block=512):
    return pl.pallas_call(
        _edge_head_body,
        grid=(E // block,),
        in_specs=[pl.BlockSpec((block, D), lambda i: (i, 0)),
                  pl.BlockSpec((D, D), lambda i: (0, 0)),
                  pl.BlockSpec((D,), lambda i: (0,)),
                  pl.BlockSpec((D, 1), lambda i: (0, 0)),
                  pl.BlockSpec((1,), lambda i: (0,))],
        out_specs=pl.BlockSpec((block,), lambda i: (i,)),
        out_shape=jax.ShapeDtypeStruct((E,), jnp.float32),
    )(ef, w1, b1, w2, b2)


def _node_mlp_body(nf_ref, agg_ref, w_ref, b_ref, o_ref, *, out_bf16):
    cc = jnp.concatenate([_bf(nf_ref[...]), _bf(agg_ref[...])], axis=1)
    o = jax.nn.relu(_dot(cc, _bf(w_ref[...])) + b_ref[...])
    o_ref[...] = _bf(o).astype(jnp.float32) if out_bf16 else o


def _node_mlp(nf, agg, w, b, out_bf16):
    return pl.pallas_call(
        functools.partial(_node_mlp_body, out_bf16=out_bf16),
        out_shape=jax.ShapeDtypeStruct((N, D), jnp.float32),
    )(nf, agg, w, b)


def _heads_body(nf_ref, wn1_ref, bn1_ref, wn2_ref, bn2_ref,
                wc1_ref, bc1_ref, wc2_ref, bc2_ref,
                we_ref, be_ref,
                pn_ref, pc_ref, ty_ref, ne_ref):
    nf = nf_ref[...]
    hn = jax.nn.relu(_dot(nf, wn1_ref[...]) + bn1_ref[...])
    pn_ref[...] = _dot(hn, wn2_ref[...]) + bn2_ref[...]
    hc = jax.nn.relu(_dot(nf, wc1_ref[...]) + bc1_ref[...])
    pc = _dot(hc, wc2_ref[...]) + bc2_ref[...]
    pc_ref[...] = pc
    mx = jnp.max(pc, axis=1, keepdims=True)
    iota = jax.lax.broadcasted_iota(jnp.int32, pc.shape, 1)
    ty = jnp.min(jnp.where(pc == mx, iota, NT), axis=1, keepdims=True)
    ty_ref[...] = ty
    ne_ref[...] = _dot(nf, we_ref[...]) + be_ref[...]


def _heads(nf, p):
    (wn1, bn1), (wn2, bn2) = p['node_cls']
    (wc1, bc1), (wc2, bc2) = p['cls']
    we, be = p['edge_const']
    return pl.pallas_call(
        _heads_body,
        out_shape=(jax.ShapeDtypeStruct((N, 1), jnp.float32),
                   jax.ShapeDtypeStruct((N, NT), jnp.float32),
                   jax.ShapeDtypeStruct((N, 1), jnp.int32),
                   jax.ShapeDtypeStruct((N, D), jnp.float32)),
    )(nf, wn1, bn1, wn2, bn2, wc1, bc1, wc2, bc2, we, be)


def _score_body(a_ref, b_ref, o_ref):
    p = a_ref[...] * b_ref[...]
    acc = p[:, 0:8]
    for j in range(1, 16):
        acc = acc + p[:, 8 * j:8 * j + 8]
    t = acc[:, 0:4] + acc[:, 4:8]
    t = t[:, 0:2] + t[:, 2:4]
    o_ref[...] = jnp.reshape(t[:, 0:1] + t[:, 1:2], (-1,))


def _scores(a, b, block=512):
    return pl.pallas_call(
        _score_body,
        grid=(E // block,),
        in_specs=[pl.BlockSpec((block, D), lambda i: (i, 0)),
                  pl.BlockSpec((block, D), lambda i: (i, 0))],
        out_specs=pl.BlockSpec((block,), lambda i: (i,)),
        out_shape=jax.ShapeDtypeStruct((E,), jnp.float32),
    )(a, b)


def _sc_gather_rows(table, idx, chunk=1000):
    n_rows, r = table.shape
    n_idx = idx.shape[0]
    per_w = n_idx // NW
    n_ch = per_w // chunk

    @functools.partial(
        pl.kernel, mesh=_mesh(),
        compiler_params=pltpu.CompilerParams(needs_layout_passes=False),
        out_type=jax.ShapeDtypeStruct((n_idx, r), table.dtype),
        scratch_types=[pltpu.VMEM((chunk,), jnp.int32),
                       pltpu.VMEM((chunk, r), table.dtype),
                       pltpu.SemaphoreType.DMA],
    )
    def k(tb, ix, out, idx_v, rows_v, sem):
        base = pl.multiple_of(_wid() * per_w, 8)

        def body(i, carry):
            off = base + i * chunk
            pltpu.sync_copy(ix.at[pl.ds(off, chunk)], idx_v)
            pltpu.async_copy(tb.at[idx_v], rows_v, sem).wait()
            pltpu.sync_copy(rows_v, out.at[pl.ds(off, chunk)])
            return carry

        lax.fori_loop(0, n_ch, body, 0)

    return k(table, idx)


def _sc_seg_kernel(ty, src, dst, chunk=2000):
    n_ch = EPW // chunk

    @functools.partial(
        pl.kernel, mesh=_mesh(),
        compiler_params=pltpu.CompilerParams(needs_layout_passes=False),
        out_type=jax.ShapeDtypeStruct((E,), jnp.int32),
        scratch_types=[pltpu.VMEM((chunk,), jnp.int32),
                       pltpu.VMEM((chunk,), jnp.int32),
                       pltpu.VMEM((chunk,), jnp.int32),
                       pltpu.VMEM((chunk,), jnp.int32),
                       pltpu.SemaphoreType.DMA],
    )
    def k(ty_h, src_h, dst_h, out_h, sv, dv, tv, ov, sem):
        base = pl.multiple_of(_wid() * EPW, 8)

        def body(i, carry):
            off = base + i * chunk
            pltpu.sync_copy(src_h.at[pl.ds(off, chunk)], sv)
            pltpu.sync_copy(dst_h.at[pl.ds(off, chunk)], dv)
            pltpu.async_copy(ty_h.at[sv], tv, sem).wait()

            def vbody(j, c):
                d16 = dv[pl.ds(j * 16, 16)]
                t16 = tv[pl.ds(j * 16, 16)]
                seg = d16 * NT + t16
                tid = (seg & 1) * TID_P + lax.shift_right_logical(seg, 1)
                ov[pl.ds(j * 16, 16)] = tid
                return c

            lax.fori_loop(0, chunk // 16, vbody, 0)
            pltpu.sync_copy(ov, out_h.at[pl.ds(off, chunk)])
            return carry

        lax.fori_loop(0, n_ch, body, 0)

    return k(ty, src, dst)


_NEG = -3.0e38


def _sc_max_partial(tidx, scores, chunk=2000):
    n_ch = (E // 16) // chunk

    @functools.partial(
        pl.kernel, mesh=_mesh(),
        compiler_params=pltpu.CompilerParams(needs_layout_passes=False),
        out_type=jax.ShapeDtypeStruct((NW * TID_P,), jnp.float32),
        scratch_types=[pltpu.VMEM((TID_P,), jnp.float32),
                       pltpu.VMEM((chunk,), jnp.int32),
                       pltpu.VMEM((chunk,), jnp.float32),
                       pltpu.SemaphoreType.DMA],
    )
    def k(tid_h, sc_h, out_h, tbl, tv, sv, sem):
        w = _wid()
        par = w % 2
        base = pl.multiple_of((w // 2) * (E // 16), 8)
        neg = jnp.full((16,), _NEG, jnp.float32)

        def initb(i, c):
            tbl[pl.ds(i * 16, 16)] = neg
            return c

        lax.fori_loop(0, TID_P // 16, initb, 0)
        lo = par * TID_P

        def body(i, carry):
            off = base + i * chunk
            pltpu.sync_copy(tid_h.at[pl.ds(off, chunk)], tv)
            pltpu.sync_copy(sc_h.at[pl.ds(off, chunk)], sv)

            def vbody(j, c):
                tid = tv[pl.ds(j * 16, 16)]
                val = sv[pl.ds(j * 16, 16)]
                iota16 = lax.iota(jnp.int32, 16)
                loc = tid - lo
                act = (loc >= 0) & (loc < TID_P)
                key = jnp.where(act, loc, TID_P)
                k, v = plsc.sort_key_val(key, val)
                for sft in (1, 2, 4, 8):
                    idxs = jnp.maximum(iota16 - sft, 0)
                    kk = k.at[idxs].get(mode="promise_in_bounds")
                    vv = v.at[idxs].get(mode="promise_in_bounds")
                    same = (kk == k) & (iota16 >= sft)
                    v = jnp.where(same, jnp.maximum(v, vv), v)
                kup = k.at[jnp.minimum(iota16 + 1, 15)].get(
                    mode="promise_in_bounds")
                m = ((k != kup) | (iota16 == 15)) & (k < TID_P)
                kc = jnp.where(m, k, 0)
                cur = plsc.load_gather(tbl, [kc], mask=m)
                new = jnp.maximum(cur, v)
                plsc.store_scatter(tbl, [kc], new, mask=m)
                return c

            lax.fori_loop(0, chunk // 16, vbody, 0)
            return carry

        lax.fori_loop(0, n_ch, body, 0)
        pltpu.sync_copy(tbl, out_h.at[pl.ds(pl.multiple_of(w * TID_P, 8), TID_P)])

    return k(tidx, scores)


def _sc_max_merge(partials):
    sl = TID_P // 16

    @functools.partial(
        pl.kernel, mesh=_mesh(),
        compiler_params=pltpu.CompilerParams(needs_layout_passes=False),
        out_type=jax.ShapeDtypeStruct((NSEG_P,), jnp.float32),
        scratch_types=[pltpu.VMEM((sl,), jnp.float32),
                       pltpu.VMEM((sl,), jnp.float32)],
    )
    def k(pt_h, out_h, acc, buf):
        w = _wid()
        par = w // 16
        rr = w % 16
        off = rr * sl
        pltpu.sync_copy(pt_h.at[pl.ds(pl.multiple_of(par * TID_P + off, 8), sl)], acc)

        def body(t, c):
            pltpu.sync_copy(pt_h.at[pl.ds(pl.multiple_of((2 * t + par) * TID_P + off, 8), sl)], buf)

            def vbody(j, cc):
                acc[pl.ds(j * 16, 16)] = jnp.maximum(acc[pl.ds(j * 16, 16)],
                                                     buf[pl.ds(j * 16, 16)])
                return cc

            lax.fori_loop(0, sl // 16, vbody, 0)
            return c

        lax.fori_loop(1, 16, body, 0)
        pltpu.sync_copy(acc, out_h.at[pl.ds(pl.multiple_of(par * TID_P + off, 8), sl)])

    return k(partials)


def _sc_ex_den(tidx, scores, mx_pm, chunk=2000):
    n_ch = EPW // chunk

    @functools.partial(
        pl.kernel, mesh=_mesh(),
        compiler_params=pltpu.CompilerParams(needs_layout_passes=False),
        out_type=(jax.ShapeDtypeStruct((E,), jnp.float32),
                  jax.ShapeDtypeStruct((2 * NSEG_P,), jnp.float32)),
        scratch_types=[pltpu.VMEM((chunk,), jnp.int32),
                       pltpu.VMEM((chunk,), jnp.float32),
                       pltpu.VMEM((chunk,), jnp.float32),
                       pltpu.VMEM((chunk,), jnp.float32),
                       pltpu.VMEM((NSEG_P // 16,), jnp.float32),
                       pltpu.VMEM_SHARED((NSEG_P,), jnp.float32),
                       pltpu.SemaphoreType.DMA],
    )
    def k(tid_h, sc_h, mx_h, ex_h, den_h, tv, sv, mv, ev, zv, shared, sem):
        w = _wid()
        core = lax.axis_index("c")
        sub = lax.axis_index("s")
        base = pl.multiple_of(w * EPW, 8)
        zn = NSEG_P // 16
        zero = jnp.zeros((16,), jnp.float32)

        def zb(i, c):
            zv[pl.ds(i * 16, 16)] = zero
            return c

        lax.fori_loop(0, zn // 16, zb, 0)
        pltpu.sync_copy(zv, shared.at[pl.ds(pl.multiple_of(sub * zn, 8), zn)])
        plsc.subcore_barrier()

        def body(i, carry):
            off = base + i * chunk
            pltpu.sync_copy(tid_h.at[pl.ds(off, chunk)], tv)
            pltpu.sync_copy(sc_h.at[pl.ds(off, chunk)], sv)
            pltpu.async_copy(mx_h.at[tv], mv, sem).wait()

            def vbody(j, c):
                s16 = sv[pl.ds(j * 16, 16)]
                m16 = mv[pl.ds(j * 16, 16)]
                ev[pl.ds(j * 16, 16)] = jnp.exp(s16 - m16)
                return c

            lax.fori_loop(0, chunk // 16, vbody, 0)
            pltpu.sync_copy(ev, ex_h.at[pl.ds(off, chunk)])
            pltpu.sync_copy(ev, shared.at[tv], add=True)
            return carry

        lax.fori_loop(0, n_ch, body, 0)
        plsc.subcore_barrier()
        pltpu.sync_copy(shared.at[pl.ds(pl.multiple_of(sub * zn, 8), zn)], zv)
        pltpu.sync_copy(zv, den_h.at[pl.ds(pl.multiple_of(core * NSEG_P + sub * zn, 8), zn)])

    return k(tidx, scores, mx_pm)


def _sc_final(tidx, ex, sig, den0, den1, chunk=2000):
    n_ch = EPW // chunk

    @functools.partial(
        pl.kernel, mesh=_mesh(),
        compiler_params=pltpu.CompilerParams(needs_layout_passes=False),
        out_type=jax.ShapeDtypeStruct((E,), jnp.float32),
        scratch_types=[pltpu.VMEM((chunk,), jnp.int32),
                       pltpu.VMEM((chunk,), jnp.float32),
                       pltpu.VMEM((chunk,), jnp.float32),
                       pltpu.VMEM((chunk,), jnp.float32),
                       pltpu.VMEM((chunk,), jnp.float32),
                       pltpu.VMEM((chunk,), jnp.float32),
                       pltpu.SemaphoreType.DMA],
    )
    def k(tid_h, ex_h, sg_h, d0_h, d1_h, out_h, tv, evv, gv, d0v, d1v, ov, sem):
        base = pl.multiple_of(_wid() * EPW, 8)

        def body(i, carry):
            off = base + i * chunk
            pltpu.sync_copy(tid_h.at[pl.ds(off, chunk)], tv)
            pltpu.sync_copy(ex_h.at[pl.ds(off, chunk)], evv)
            pltpu.sync_copy(sg_h.at[pl.ds(off, chunk)], gv)
            pltpu.async_copy(d0_h.at[tv], d0v, sem).wait()
            pltpu.async_copy(d1_h.at[tv], d1v, sem).wait()

            def vbody(j, c):
                e16 = evv[pl.ds(j * 16, 16)]
                g16 = gv[pl.ds(j * 16, 16)]
                d16 = d0v[pl.ds(j * 16, 16)] + d1v[pl.ds(j * 16, 16)]
                ov[pl.ds(j * 16, 16)] = e16 / d16 * g16
                return c

            lax.fori_loop(0, chunk // 16, vbody, 0)
            pltpu.sync_copy(ov, out_h.at[pl.ds(off, chunk)])
            return carry

        lax.fori_loop(0, n_ch, body, 0)

    return k(tidx, ex, sig, den0, den1)


def kernel(x, edge_attr, params, edge_index, node_types):
    src = edge_index[0]
    dst = edge_index[1]
    (wne1, bne1), (wne2, bne2) = params['node_emb']
    (wee1, bee1), (wee2, bee2) = params['edge_emb']
    (w1, b1), (w2, b2) = params['mpn_edge']
    (wn, bn), = params['mpn_node']

    nf = _node_emb(x, wne1, bne1, wne2, bne2)
    ef = _edge_emb(edge_attr, wee1, bee1, wee2, bee2)

    for step in range(STEPS):
        gs = _sc_gather_rows(nf, src)
        gd = _sc_gather_rows(nf, dst)
        ef = _edge_mlp(gs, gd, ef, w1, b1, w2, b2)
        agg = jax.ops.segment_sum(ef, dst, num_segments=N)
        nf = _node_mlp(nf, agg, wn, bn, out_bf16=(step == 0))

    pn, pc, ty, ne = _heads(nf, params)
    pred_node = pn[:, 0]
    pred_class = pc
    (we1, be1), (we2, be2) = params['edge_cls']
    sig = jax.nn.sigmoid(_edge_head(ef, we1, be1, we2, be2))

    nes = _sc_gather_rows(ne, src)
    ned = _sc_gather_rows(ne, dst)
    scores = _scores(nes, ned)

    tidx = _sc_seg_kernel(ty[:, 0], src, dst)
    parts = _sc_max_partial(tidx, scores)
    mx_pm = _sc_max_merge(parts)
    ex, den = _sc_ex_den(tidx, scores, mx_pm)
    pred_edge = _sc_final(tidx, ex, sig, den[:NSEG_P], den[NSEG_P:])
    return ([pred_edge], [pred_node], [pred_class], nf, ef)
